# baseline (speedup 1.0000x reference)
import functools

import jax
import jax.numpy as jnp
from jax import lax
from jax.experimental import pallas as pl
from jax.experimental.pallas import tpu as pltpu
from jax.experimental.pallas import tpu_sc as plsc

N = 10000
D = 128
E = 320000
NT = 32
CHUNK = 128
NCH = 80
NCH0 = 80
NCH1 = 80
EP = NT * NCH * CHUNK
NPAD = 10240
RPT = NPAD // 16
DUMMY = N

_f32 = jnp.float32


def _wid(c, s):
    return s * 2 + c


def _deg_body(src_hbm, dst_hbm, dego_hbm, degi_hbm,
              ia, ib, cnto, cnti, sem_i):
    c = lax.axis_index("c")
    s = lax.axis_index("s")
    w = _wid(c, s)
    zero16 = jnp.zeros((16,), _f32)
    one16 = jnp.ones((16,), _f32)

    ca = pltpu.async_copy(src_hbm.at[pl.ds(w * NCH, NCH)], ia, sem_i)
    cb = pltpu.async_copy(dst_hbm.at[pl.ds(w * NCH, NCH)], ib, sem_i)

    def zcnt(i, carry):
        cnto[pl.ds(i * 16, 16)] = zero16
        cnti[pl.ds(i * 16, 16)] = zero16
        return carry

    lax.fori_loop(0, NPAD // 16, zcnt, 0)
    ca.wait()
    cb.wait()

    for idx_v, cnt in ((ia, cnto), (ib, cnti)):
        def body(j, carry):
            for k in range(CHUNK // 16):
                plsc.addupdate_scatter(cnt, [idx_v[j, pl.ds(k * 16, 16)]], one16)
            return carry

        lax.fori_loop(0, NCH, body, 0)

    pltpu.sync_copy(cnto, dego_hbm.at[w])
    pltpu.sync_copy(cnti, degi_hbm.at[w])


_deg_call = functools.partial(
    pl.kernel,
    out_type=[
        jax.ShapeDtypeStruct((NT, NPAD), _f32),
        jax.ShapeDtypeStruct((NT, NPAD), _f32),
    ],
    mesh=plsc.VectorSubcoreMesh(core_axis_name="c", subcore_axis_name="s",
                                num_cores=2, num_subcores=16),
    scratch_types=[
        pltpu.VMEM((NCH, CHUNK), jnp.int32),
        pltpu.VMEM((NCH, CHUNK), jnp.int32),
        pltpu.VMEM((NPAD,), _f32),
        pltpu.VMEM((NPAD,), _f32),
        pltpu.SemaphoreType.DMA,
    ],
    compiler_params=pltpu.CompilerParams(needs_layout_passes=False),
)


def _agg_body(h_hbm, src_hbm, dst_hbm, out_hbm,
              s0, s1, d0, d1, d2, rows_a, rows_b, agg_sh,
              sem_i, sem_g, sem_s):
    c = lax.axis_index("c")
    s = lax.axis_index("s")
    w = _wid(c, s)
    rows = [rows_a, rows_b]
    sv = [s0, s1]
    dv = [d0, d1, d2]
    zero16 = jnp.zeros((16,), _f32)

    def zrows(t, carry):
        rows_a[t // 8, pl.ds((t % 8) * 16, 16)] = zero16
        return carry

    lax.fori_loop(0, CHUNK * 8, zrows, 0)
    for k in range(RPT // CHUNK):
        pltpu.sync_copy(rows_a, agg_sh.at[pl.ds(s * RPT + k * CHUNK, CHUNK)])
    plsc.subcore_barrier()

    base = w * NCH

    def drain64k(sem):
        pltpu.make_async_copy(h_hbm.at[pl.ds(0, CHUNK)], rows_a, sem).wait()

    def drain_idx_pair():
        pltpu.make_async_copy(src_hbm.at[base], s0, sem_i).wait()
        pltpu.make_async_copy(src_hbm.at[base], s0, sem_i).wait()

    def idx_load(t2, su, du):
        tcl = lax.min(t2, NCH - 1)
        pltpu.async_copy(src_hbm.at[base + tcl], sv[su], sem_i)
        pltpu.async_copy(dst_hbm.at[base + tcl], dv[du], sem_i)

    def step(t, phase):
        drain64k(sem_g)
        pltpu.async_copy(rows[phase % 2], agg_sh.at[dv[phase % 3]],
                         sem_s, add=True)
        drain64k(sem_s)
        drain_idx_pair()
        idx_load(t + 2, (phase + 2) % 2, (phase + 2) % 3)
        pltpu.async_copy(h_hbm.at[sv[(phase + 1) % 2]],
                         rows[(phase + 1) % 2], sem_g)

    pltpu.sync_copy(src_hbm.at[base], s0)
    pltpu.sync_copy(dst_hbm.at[base], d0)
    pltpu.async_copy(h_hbm.at[s0], rows_a, sem_g)
    idx_load(1, 1, 1)
    drain64k(sem_g)
    pltpu.async_copy(rows_a, agg_sh.at[d0], sem_s, add=True)
    drain_idx_pair()
    idx_load(2, 0, 2)
    pltpu.async_copy(h_hbm.at[s1], rows_b, sem_g)

    def group(p, carry):
        t0 = 1 + p * 6
        for u in range(6):
            step(t0 + u, 1 + u)
        return carry

    lax.fori_loop(0, (NCH - 2) // 6, group, 0)
    drain64k(sem_g)
    pltpu.async_copy(rows[1], agg_sh.at[dv[1]], sem_s, add=True)
    drain64k(sem_s)
    drain_idx_pair()
    drain64k(sem_s)
    plsc.subcore_barrier()

    sl = pl.ds(s * RPT, RPT)
    pltpu.sync_copy(agg_sh.at[sl], out_hbm.at[c, sl])


_agg_call = functools.partial(
    pl.kernel,
    out_type=jax.ShapeDtypeStruct((2, NPAD, D), _f32),
    mesh=plsc.VectorSubcoreMesh(core_axis_name="c", subcore_axis_name="s",
                                num_cores=2, num_subcores=16),
    scratch_types=[
        pltpu.VMEM((CHUNK,), jnp.int32),
        pltpu.VMEM((CHUNK,), jnp.int32),
        pltpu.VMEM((CHUNK,), jnp.int32),
        pltpu.VMEM((CHUNK,), jnp.int32),
        pltpu.VMEM((CHUNK,), jnp.int32),
        pltpu.VMEM((CHUNK, D), _f32),
        pltpu.VMEM((CHUNK, D), _f32),
        pltpu.VMEM_SHARED((NPAD, D), _f32),
        pltpu.SemaphoreType.DMA,
        pltpu.SemaphoreType.DMA,
        pltpu.SemaphoreType.DMA,
    ],
)


def _norm_col(degt_ref):
    deg = jnp.sum(degt_ref[...], axis=1, keepdims=True)
    nrm = jnp.where(deg > 0, lax.rsqrt(jnp.maximum(deg, 1.0)), 0.0)
    return nrm[:N]


def _tc1_body(x_ref, w_ref, dego_ref, h_ref):
    h = jnp.dot(x_ref[...], w_ref[...], preferred_element_type=_f32)
    h_ref[:N] = h * _norm_col(dego_ref)
    h_ref[N:] = jnp.zeros((NPAD - N, D), _f32)


def _tc2_body(parts_ref, degi_ref, dego_ref, b_ref, w_ref, h_ref):
    agg = parts_ref[0, :N] + parts_ref[1, :N]
    out1 = jnp.maximum(agg * _norm_col(degi_ref) + b_ref[...], 0.0)
    h2 = jnp.dot(out1, w_ref[...], preferred_element_type=_f32)
    h_ref[:N] = h2 * _norm_col(dego_ref)
    h_ref[N:] = jnp.zeros((NPAD - N, D), _f32)


def _tc3_body(parts_ref, degi_ref, b_ref, o_ref):
    agg = parts_ref[0, :N] + parts_ref[1, :N]
    o_ref[...] = jnp.maximum(agg * _norm_col(degi_ref) + b_ref[...], 0.0)


def kernel(x, edge_index, W1, b1, W2, b2):
    src = edge_index[0].astype(jnp.int32)
    dst = edge_index[1].astype(jnp.int32)
    pad = jnp.tile(DUMMY + jnp.arange(CHUNK, dtype=jnp.int32), (EP - E) // CHUNK)
    srcp = jnp.concatenate([src, pad]).reshape(NT * NCH, CHUNK)
    dstp = jnp.concatenate([dst, pad]).reshape(NT * NCH, CHUNK)
    b1r = b1.reshape(1, D)
    b2r = b2.reshape(1, D)

    dego_p, degi_p = _deg_call(_deg_body)(srcp, dstp)
    dego = dego_p.T
    degi = degi_p.T

    h1 = pl.pallas_call(
        _tc1_body, out_shape=jax.ShapeDtypeStruct((NPAD, D), _f32),
    )(x, W1, dego)

    parts1 = _agg_call(_agg_body)(h1, srcp, dstp)

    h2 = pl.pallas_call(
        _tc2_body, out_shape=jax.ShapeDtypeStruct((NPAD, D), _f32),
    )(parts1, degi, dego, b1r, W2)

    parts2 = _agg_call(_agg_body)(h2, srcp, dstp)

    out = pl.pallas_call(
        _tc3_body, out_shape=jax.ShapeDtypeStruct((N, D), _f32),
    )(parts2, degi, b2r)
    return out


# speedup vs baseline: 1.0443x; 1.0443x over previous
import functools

import jax
import jax.numpy as jnp
from jax import lax
from jax.experimental import pallas as pl
from jax.experimental.pallas import tpu as pltpu
from jax.experimental.pallas import tpu_sc as plsc

N = 10000
D = 128
E = 320000
NT = 32
CHUNK = 128
NCH = 80
NCH0 = 80
NCH1 = 80
EP = NT * NCH * CHUNK
NPAD = 10240
RPT = NPAD // 16
DUMMY = N

_f32 = jnp.float32


def _wid(c, s):
    return s * 2 + c


def _deg_body(src_hbm, dst_hbm, dego_hbm, degi_hbm,
              ia, ib, cnto, cnti, sem_i):
    c = lax.axis_index("c")
    s = lax.axis_index("s")
    w = _wid(c, s)
    zero16 = jnp.zeros((16,), _f32)
    one16 = jnp.ones((16,), _f32)

    ca = pltpu.async_copy(src_hbm.at[pl.ds(w * NCH, NCH)], ia, sem_i)
    cb = pltpu.async_copy(dst_hbm.at[pl.ds(w * NCH, NCH)], ib, sem_i)

    def zcnt(i, carry):
        cnto[pl.ds(i * 16, 16)] = zero16
        cnti[pl.ds(i * 16, 16)] = zero16
        return carry

    lax.fori_loop(0, NPAD // 16, zcnt, 0)
    ca.wait()
    cb.wait()

    for idx_v, cnt in ((ia, cnto), (ib, cnti)):
        def body(j, carry):
            for k in range(CHUNK // 16):
                plsc.addupdate_scatter(cnt, [idx_v[j, pl.ds(k * 16, 16)]], one16)
            return carry

        lax.fori_loop(0, NCH, body, 0)

    pltpu.sync_copy(cnto, dego_hbm.at[w])
    pltpu.sync_copy(cnti, degi_hbm.at[w])


_deg_call = functools.partial(
    pl.kernel,
    out_type=[
        jax.ShapeDtypeStruct((NT, NPAD), _f32),
        jax.ShapeDtypeStruct((NT, NPAD), _f32),
    ],
    mesh=plsc.VectorSubcoreMesh(core_axis_name="c", subcore_axis_name="s",
                                num_cores=2, num_subcores=16),
    scratch_types=[
        pltpu.VMEM((NCH, CHUNK), jnp.int32),
        pltpu.VMEM((NCH, CHUNK), jnp.int32),
        pltpu.VMEM((NPAD,), _f32),
        pltpu.VMEM((NPAD,), _f32),
        pltpu.SemaphoreType.DMA,
    ],
    compiler_params=pltpu.CompilerParams(needs_layout_passes=False),
)


def _agg_body(h_hbm, src_hbm, dst_hbm, out_hbm,
              sa, sb, da, db, rows_a, rows_b, agg_sh,
              sem_i, sem_a, sem_b):
    c = lax.axis_index("c")
    s = lax.axis_index("s")
    w = _wid(c, s)
    zero16 = jnp.zeros((16,), _f32)

    def zrows(t, carry):
        rows_a[t // 8, pl.ds((t % 8) * 16, 16)] = zero16
        return carry

    lax.fori_loop(0, CHUNK * 8, zrows, 0)
    for k in range(RPT // CHUNK):
        pltpu.sync_copy(rows_a, agg_sh.at[pl.ds(s * RPT + k * CHUNK, CHUNK)])
    plsc.subcore_barrier()

    base = jnp.where(c == 0, s * NCH0, (NT // 2) * NCH0 + s * NCH1)
    nch = jnp.where(c == 0, NCH0, NCH1)

    pltpu.sync_copy(src_hbm.at[base], sa)
    pltpu.sync_copy(dst_hbm.at[base], da)
    ga0 = pltpu.async_copy(h_hbm.at[sa], rows_a, sem_a)

    def body(p, carry):
        j = p * 2
        cs = pltpu.async_copy(src_hbm.at[base + j + 1], sb, sem_i)
        cd = pltpu.async_copy(dst_hbm.at[base + j + 1], db, sem_i)
        cs.wait()
        cd.wait()
        gb = pltpu.async_copy(h_hbm.at[sb], rows_b, sem_b)
        pltpu.make_async_copy(h_hbm.at[sa], rows_a, sem_a).wait()
        pltpu.sync_copy(rows_a, agg_sh.at[da], add=True)
        jn = lax.min(j + 2, nch - 1)
        cs2 = pltpu.async_copy(src_hbm.at[base + jn], sa, sem_i)
        cd2 = pltpu.async_copy(dst_hbm.at[base + jn], da, sem_i)
        cs2.wait()
        cd2.wait()
        ga = pltpu.async_copy(h_hbm.at[sa], rows_a, sem_a)
        gb.wait()
        pltpu.sync_copy(rows_b, agg_sh.at[db], add=True)
        return carry

    lax.fori_loop(0, nch // 2, body, 0)
    pltpu.make_async_copy(h_hbm.at[sa], rows_a, sem_a).wait()
    plsc.subcore_barrier()

    sl = pl.ds(s * RPT, RPT)
    pltpu.sync_copy(agg_sh.at[sl], out_hbm.at[c, sl])


_agg_call = functools.partial(
    pl.kernel,
    out_type=jax.ShapeDtypeStruct((2, NPAD, D), _f32),
    mesh=plsc.VectorSubcoreMesh(core_axis_name="c", subcore_axis_name="s",
                                num_cores=2, num_subcores=16),
    scratch_types=[
        pltpu.VMEM((CHUNK,), jnp.int32),
        pltpu.VMEM((CHUNK,), jnp.int32),
        pltpu.VMEM((CHUNK,), jnp.int32),
        pltpu.VMEM((CHUNK,), jnp.int32),
        pltpu.VMEM((CHUNK, D), _f32),
        pltpu.VMEM((CHUNK, D), _f32),
        pltpu.VMEM_SHARED((NPAD, D), _f32),
        pltpu.SemaphoreType.DMA,
        pltpu.SemaphoreType.DMA,
        pltpu.SemaphoreType.DMA,
    ],
)


def _norm_col(degt_ref):
    deg = jnp.sum(degt_ref[...], axis=1, keepdims=True)
    nrm = jnp.where(deg > 0, lax.rsqrt(jnp.maximum(deg, 1.0)), 0.0)
    return nrm[:N]


def _tc1_body(x_ref, w_ref, dego_ref, h_ref):
    h = jnp.dot(x_ref[...], w_ref[...], preferred_element_type=_f32)
    h_ref[:N] = h * _norm_col(dego_ref)
    h_ref[N:] = jnp.zeros((NPAD - N, D), _f32)


def _tc2_body(parts_ref, degi_ref, dego_ref, b_ref, w_ref, h_ref):
    agg = parts_ref[0, :N] + parts_ref[1, :N]
    out1 = jnp.maximum(agg * _norm_col(degi_ref) + b_ref[...], 0.0)
    h2 = jnp.dot(out1, w_ref[...], preferred_element_type=_f32)
    h_ref[:N] = h2 * _norm_col(dego_ref)
    h_ref[N:] = jnp.zeros((NPAD - N, D), _f32)


def _tc3_body(parts_ref, degi_ref, b_ref, o_ref):
    agg = parts_ref[0, :N] + parts_ref[1, :N]
    o_ref[...] = jnp.maximum(agg * _norm_col(degi_ref) + b_ref[...], 0.0)


def kernel(x, edge_index, W1, b1, W2, b2):
    src = edge_index[0].astype(jnp.int32)
    dst = edge_index[1].astype(jnp.int32)
    pad = jnp.tile(DUMMY + jnp.arange(CHUNK, dtype=jnp.int32), (EP - E) // CHUNK)
    srcp = jnp.concatenate([src, pad]).reshape(NT * NCH, CHUNK)
    dstp = jnp.concatenate([dst, pad]).reshape(NT * NCH, CHUNK)
    b1r = b1.reshape(1, D)
    b2r = b2.reshape(1, D)

    dego_p, degi_p = _deg_call(_deg_body)(srcp, dstp)
    dego = dego_p.T
    degi = degi_p.T

    h1 = pl.pallas_call(
        _tc1_body, out_shape=jax.ShapeDtypeStruct((NPAD, D), _f32),
    )(x, W1, dego)

    parts1 = _agg_call(_agg_body)(h1, srcp, dstp)

    h2 = pl.pallas_call(
        _tc2_body, out_shape=jax.ShapeDtypeStruct((NPAD, D), _f32),
    )(parts1, degi, dego, b1r, W2)

    parts2 = _agg_call(_agg_body)(h2, srcp, dstp)

    out = pl.pallas_call(
        _tc3_body, out_shape=jax.ShapeDtypeStruct((N, D), _f32),
    )(parts2, degi, b2r)
    return out
